# Initial kernel scaffold; baseline (speedup 1.0000x reference)
import functools
import math

import numpy as np
import jax
import jax.numpy as jnp
from jax import lax
from jax.experimental import pallas as pl
from jax.experimental.pallas import tpu as pltpu
from jax.experimental.pallas import tpu_sc as plsc

_N = 512
_C = 96
_NBANDS = 32
_BAND = _N // _NBANDS
_PX = _BAND * _N
_W = 44
_WPX = _W * _N
_NC = 2
_NS = 16

_CUTMIN, _CUTMAX, _ALPHA = 2, 32, 1.0


def _build_field():
    n = _N
    beta_sample = 0.5
    cut = int(beta_sample * (_CUTMAX + 1 - _CUTMIN) + _CUTMIN)
    c_ = cut + 1e-06
    lg = math.log(c_)
    t1 = 1.0 / (math.pi * n ** 2 * lg)
    t2 = 4.0 / (math.pi ** 3 * c_ ** 2 * lg)
    t2 = max(t1, _ALPHA * t2)
    t = beta_sample * (t2 - t1) + t1

    def field(m, key):
        x = jnp.linspace(0.0, 1.0, n, dtype=jnp.float32)
        k = jnp.arange(1, m + 1, dtype=jnp.float32)
        i, j = jnp.meshgrid(k, k, indexing='ij')
        r = jnp.sqrt(i ** 2 + j ** 2)
        e = (r < m + 0.5).astype(jnp.float32) / r
        s = jnp.sin(jnp.pi * x[:, None] * k[None, :])
        c = jax.random.normal(key, (m, m), dtype=jnp.float32) * e
        return jnp.einsum('ij,xi,yj->yx', c, s, s)

    ku, kv = jax.random.split(jax.random.key(1))
    dx = (t ** 0.5) * field(cut, ku) * n
    dy = (t ** 0.5) * field(cut, kv) * n
    y, x = jnp.meshgrid(jnp.arange(n, dtype=jnp.float32),
                        jnp.arange(n, dtype=jnp.float32), indexing='ij')
    xn = jnp.clip(x - dx, 0.0, n - 1)
    yn = jnp.clip(y - dy, 0.0, n - 1)
    ybf = jnp.clip(jnp.floor(yn), 0, n - 2)
    xbf = jnp.clip(jnp.floor(xn), 0, n - 2)
    yv = yn - ybf
    xv = xn - xbf
    gbase = ybf.astype(jnp.int32) * n + xbf.astype(jnp.int32)
    return gbase, xv, yv


_CONST_CACHE = {}


def _consts():
    if not _CONST_CACHE:
        gbase, xv, yv = [np.asarray(a) for a in jax.jit(_build_field)()]
        ybrow = gbase // _N
        lo = np.zeros(_NBANDS, np.int32)
        for w in range(_NBANDS):
            blk = ybrow[w * _BAND:(w + 1) * _BAND]
            l = int(min(max(int(blk.min()) - 2, 0), _N - _W))
            assert int(blk.max()) + 1 <= l + _W - 1, "window too narrow"
            lo[w] = l
        band_of_row = np.arange(_N) // _BAND
        base_loc = gbase - (lo[band_of_row] * _N)[:, None]
        _CONST_CACHE["base"] = jnp.asarray(
            base_loc.reshape(_NBANDS, _PX).astype(np.int32))
        _CONST_CACHE["xv"] = jnp.asarray(xv.reshape(_NBANDS, _PX))
        _CONST_CACHE["yv"] = jnp.asarray(yv.reshape(_NBANDS, _PX))
        _CONST_CACHE["lo512"] = jnp.asarray((lo.astype(np.int32)) * _N)
    return (_CONST_CACHE["base"], _CONST_CACHE["xv"], _CONST_CACHE["yv"],
            _CONST_CACHE["lo512"])


def _make_remap():
    mesh = plsc.VectorSubcoreMesh(core_axis_name="c", subcore_axis_name="s")

    @functools.partial(
        pl.kernel,
        out_type=jax.ShapeDtypeStruct((_C, _N * _N), jnp.float32),
        mesh=mesh,
        scratch_types=[
            pltpu.VMEM((_PX,), jnp.int32),
            pltpu.VMEM((_PX,), jnp.float32),
            pltpu.VMEM((_PX,), jnp.float32),
            pltpu.VMEM((_NBANDS,), jnp.int32),
            pltpu.VMEM((2 * _WPX,), jnp.float32),
            pltpu.VMEM((2 * _PX,), jnp.float32),
            pltpu.SemaphoreType.DMA,
            pltpu.SemaphoreType.DMA,
            pltpu.SemaphoreType.DMA,
            pltpu.SemaphoreType.DMA,
        ],
    )
    def _remap(img_hbm, base_hbm, xv_hbm, yv_hbm, lo_hbm, out_hbm,
               base_v, xv_v, yv_v, lo_v, win_v, out_v,
               sem_in0, sem_in1, sem_out0, sem_out1):
        cid = lax.axis_index("c")
        sid = lax.axis_index("s")
        wid = sid * _NC + cid

        pltpu.sync_copy(base_hbm.at[wid], base_v)
        pltpu.sync_copy(xv_hbm.at[wid], xv_v)
        pltpu.sync_copy(yv_hbm.at[wid], yv_v)
        pltpu.sync_copy(lo_hbm, lo_v)
        lo512 = lo_v[wid]
        obase = wid * _PX

        sems_in = (sem_in0, sem_in1)
        sems_out = (sem_out0, sem_out1)

        def in_copy(c, p):
            return pltpu.make_async_copy(
                img_hbm.at[c, pl.ds(lo512, _WPX)],
                win_v.at[pl.ds(p * _WPX, _WPX)],
                sems_in[p])

        def out_copy(c, p):
            return pltpu.make_async_copy(
                out_v.at[pl.ds(p * _PX, _PX)],
                out_hbm.at[c, pl.ds(obase, _PX)],
                sems_out[p])

        in_copy(0, 0).start()

        def compute(p):
            woff = p * _WPX

            def px(i, _):
                sl = pl.ds(i * 16, 16)
                b = base_v[sl] + woff
                xv = xv_v[sl]
                yv = yv_v[sl]
                g00 = plsc.load_gather(win_v, [b])
                g01 = plsc.load_gather(win_v, [b + 1])
                g10 = plsc.load_gather(win_v, [b + _N])
                g11 = plsc.load_gather(win_v, [b + (_N + 1)])
                top = g00 + xv * (g01 - g00)
                bot = g10 + xv * (g11 - g10)
                out_v[pl.ds(p * _PX + i * 16, 16)] = top + yv * (bot - top)
                return 0

            lax.fori_loop(0, _PX // 16, px, 0, unroll=4)

        def chan_pair(c2, _):
            for p in (0, 1):
                c = c2 * 2 + p
                in_copy(c, p).wait()

                @pl.when(c < _C - 1)
                def _start_next():
                    in_copy(c + 1, 1 - p).start()

                @pl.when(c >= 2)
                def _free_out():
                    out_copy(c - 2, p).wait()

                compute(p)
                out_copy(c, p).start()
            return 0

        lax.fori_loop(0, _C // 2, chan_pair, 0)
        out_copy(_C - 2, 0).wait()
        out_copy(_C - 1, 1).wait()

    return _remap


_REMAP = _make_remap()


def kernel(img):
    base, xv, yv, lo512 = _consts()
    img2 = img.reshape(_C, _N * _N)
    out = _REMAP(img2, base, xv, yv, lo512)
    return out.reshape(img.shape)


_consts()


# speedup vs baseline: 68.2560x; 68.2560x over previous
import functools
import math

import jax
import jax.numpy as jnp
from jax import lax
from jax.experimental import pallas as pl
from jax.experimental.pallas import tpu as pltpu
from jax.experimental.pallas import tpu_sc as plsc

_N = 512
_C = 96
_NBANDS = 32
_BAND = _N // _NBANDS
_PX = _BAND * _N
_W = 44
_WPX = _W * _N
_NC = 2
_NS = 16

_CUTMIN, _CUTMAX, _ALPHA = 2, 32, 1.0


def _build_field():
    n = _N
    beta_sample = 0.5
    cut = int(beta_sample * (_CUTMAX + 1 - _CUTMIN) + _CUTMIN)
    c_ = cut + 1e-06
    lg = math.log(c_)
    t1 = 1.0 / (math.pi * n ** 2 * lg)
    t2 = 4.0 / (math.pi ** 3 * c_ ** 2 * lg)
    t2 = max(t1, _ALPHA * t2)
    t = beta_sample * (t2 - t1) + t1

    def field(m, key):
        x = jnp.linspace(0.0, 1.0, n, dtype=jnp.float32)
        k = jnp.arange(1, m + 1, dtype=jnp.float32)
        i, j = jnp.meshgrid(k, k, indexing='ij')
        r = jnp.sqrt(i ** 2 + j ** 2)
        e = (r < m + 0.5).astype(jnp.float32) / r
        s = jnp.sin(jnp.pi * x[:, None] * k[None, :])
        c = jax.random.normal(key, (m, m), dtype=jnp.float32) * e
        return jnp.einsum('ij,xi,yj->yx', c, s, s)

    ku, kv = jax.random.split(jax.random.key(1))
    dx = (t ** 0.5) * field(cut, ku) * n
    dy = (t ** 0.5) * field(cut, kv) * n
    y, x = jnp.meshgrid(jnp.arange(n, dtype=jnp.float32),
                        jnp.arange(n, dtype=jnp.float32), indexing='ij')
    xn = jnp.clip(x - dx, 0.0, n - 1)
    yn = jnp.clip(y - dy, 0.0, n - 1)
    ybf = jnp.clip(jnp.floor(yn), 0, n - 2)
    xbf = jnp.clip(jnp.floor(xn), 0, n - 2)
    yv = yn - ybf
    xv = xn - xbf
    gbase = ybf.astype(jnp.int32) * n + xbf.astype(jnp.int32)
    return gbase, xv, yv


def _consts():
    gbase, xv, yv = _build_field()
    ybrow = gbase // _N
    ybmin = jnp.min(ybrow.reshape(_NBANDS, _PX), axis=1)
    lo = jnp.clip(ybmin - 2, 0, _N - _W).astype(jnp.int32)
    lo512 = lo * _N
    base = gbase.reshape(_NBANDS, _PX) - lo512[:, None]
    lo512_pad = jnp.concatenate([lo512, jnp.zeros(16, jnp.int32)])
    return base.reshape(-1), xv.reshape(-1), yv.reshape(-1), lo512_pad


def _make_remap():
    mesh = plsc.VectorSubcoreMesh(core_axis_name="c", subcore_axis_name="s")

    @functools.partial(
        pl.kernel,
        out_type=jax.ShapeDtypeStruct((_C * _N * _N,), jnp.float32),
        mesh=mesh,
        compiler_params=pltpu.CompilerParams(needs_layout_passes=False),
        scratch_types=[
            pltpu.VMEM((_PX,), jnp.int32),
            pltpu.VMEM((_PX,), jnp.float32),
            pltpu.VMEM((_PX,), jnp.float32),
            pltpu.VMEM((_NBANDS + 16,), jnp.int32),
            pltpu.VMEM((2 * _WPX,), jnp.float32),
            pltpu.VMEM((2 * _PX,), jnp.float32),
            pltpu.SemaphoreType.DMA,
            pltpu.SemaphoreType.DMA,
            pltpu.SemaphoreType.DMA,
            pltpu.SemaphoreType.DMA,
        ],
    )
    def _remap(img_hbm, base_hbm, xv_hbm, yv_hbm, lo_hbm, out_hbm,
               base_v, xv_v, yv_v, lo_v, win_v, out_v,
               sem_in0, sem_in1, sem_out0, sem_out1):
        cid = lax.axis_index("c")
        sid = lax.axis_index("s")
        wid = sid * _NC + cid
        mband = pl.multiple_of(wid * _PX, _PX)

        pltpu.sync_copy(base_hbm.at[pl.ds(mband, _PX)], base_v)
        pltpu.sync_copy(xv_hbm.at[pl.ds(mband, _PX)], xv_v)
        pltpu.sync_copy(yv_hbm.at[pl.ds(mband, _PX)], yv_v)
        pltpu.sync_copy(lo_hbm, lo_v)
        lo512 = pl.multiple_of(lo_v[pl.ds(wid, 16)][0], _N)

        sems_in = (sem_in0, sem_in1)
        sems_out = (sem_out0, sem_out1)

        def in_copy(c, p):
            return pltpu.make_async_copy(
                img_hbm.at[pl.ds(pl.multiple_of(c * (_N * _N) + lo512, _N),
                                 _WPX)],
                win_v.at[pl.ds(p * _WPX, _WPX)],
                sems_in[p])

        def out_copy(c, p):
            return pltpu.make_async_copy(
                out_v.at[pl.ds(p * _PX, _PX)],
                out_hbm.at[pl.ds(pl.multiple_of(c * (_N * _N) + mband, _PX),
                                 _PX)],
                sems_out[p])

        in_copy(0, 0).start()

        def compute(p):
            woff = p * _WPX

            def px(i, _):
                sl = pl.ds(i * 16, 16)
                b = base_v[sl] + woff
                xv = xv_v[sl]
                yv = yv_v[sl]
                g00 = plsc.load_gather(win_v, [b])
                g01 = plsc.load_gather(win_v, [b + 1])
                g10 = plsc.load_gather(win_v, [b + _N])
                g11 = plsc.load_gather(win_v, [b + (_N + 1)])
                top = g00 + xv * (g01 - g00)
                bot = g10 + xv * (g11 - g10)
                out_v[pl.ds(p * _PX + i * 16, 16)] = top + yv * (bot - top)
                return 0

            lax.fori_loop(0, _PX // 16, px, 0, unroll=4)

        def chan_pair(c2, _):
            for p in (0, 1):
                c = c2 * 2 + p
                in_copy(c, p).wait()

                @pl.when(c < _C - 1)
                def _start_next():
                    in_copy(c + 1, 1 - p).start()

                @pl.when(c >= 2)
                def _free_out():
                    out_copy(c - 2, p).wait()

                compute(p)
                out_copy(c, p).start()
            return 0

        lax.fori_loop(0, _C // 2, chan_pair, 0)
        out_copy(_C - 2, 0).wait()
        out_copy(_C - 1, 1).wait()

    return _remap


_REMAP_CACHE = []


def kernel(img):
    base, xv, yv, lo512 = _consts()
    if not _REMAP_CACHE:
        _REMAP_CACHE.append(_make_remap())
    img1 = img.reshape(-1)
    out = _REMAP_CACHE[0](img1, base, xv, yv, lo512)
    return out.reshape(img.shape)


# speedup vs baseline: 179.7867x; 2.6340x over previous
import functools
import math

import jax
import jax.numpy as jnp
from jax import lax
from jax.experimental import pallas as pl
from jax.experimental.pallas import tpu as pltpu
from jax.experimental.pallas import tpu_sc as plsc

_N = 512
_C = 96
_NBANDS = 32
_BAND = _N // _NBANDS
_PX = _BAND * _N
_W = 44
_WPX = _W * _N
_NC = 2
_NS = 16

_CUTMIN, _CUTMAX, _ALPHA = 2, 32, 1.0


def _build_field():
    n = _N
    beta_sample = 0.5
    cut = int(beta_sample * (_CUTMAX + 1 - _CUTMIN) + _CUTMIN)
    c_ = cut + 1e-06
    lg = math.log(c_)
    t1 = 1.0 / (math.pi * n ** 2 * lg)
    t2 = 4.0 / (math.pi ** 3 * c_ ** 2 * lg)
    t2 = max(t1, _ALPHA * t2)
    t = beta_sample * (t2 - t1) + t1

    def field(m, key):
        x = jnp.linspace(0.0, 1.0, n, dtype=jnp.float32)
        k = jnp.arange(1, m + 1, dtype=jnp.float32)
        i, j = jnp.meshgrid(k, k, indexing='ij')
        r = jnp.sqrt(i ** 2 + j ** 2)
        e = (r < m + 0.5).astype(jnp.float32) / r
        s = jnp.sin(jnp.pi * x[:, None] * k[None, :])
        c = jax.random.normal(key, (m, m), dtype=jnp.float32) * e
        return jnp.einsum('ij,xi,yj->yx', c, s, s)

    ku, kv = jax.random.split(jax.random.key(1))
    dx = (t ** 0.5) * field(cut, ku) * n
    dy = (t ** 0.5) * field(cut, kv) * n
    y, x = jnp.meshgrid(jnp.arange(n, dtype=jnp.float32),
                        jnp.arange(n, dtype=jnp.float32), indexing='ij')
    xn = jnp.clip(x - dx, 0.0, n - 1)
    yn = jnp.clip(y - dy, 0.0, n - 1)
    ybf = jnp.clip(jnp.floor(yn), 0, n - 2)
    xbf = jnp.clip(jnp.floor(xn), 0, n - 2)
    yv = yn - ybf
    xv = xn - xbf
    gbase = ybf.astype(jnp.int32) * n + xbf.astype(jnp.int32)
    return gbase, xv, yv


def _consts():
    gbase, xv, yv = _build_field()
    ybrow = gbase // _N
    ybmin = jnp.min(ybrow.reshape(_NBANDS, _PX), axis=1)
    lo = jnp.clip(ybmin - 2, 0, _N - _W).astype(jnp.int32)
    lo512 = lo * _N
    base = gbase.reshape(_NBANDS, _PX) - lo512[:, None]
    lo512_pad = jnp.concatenate([lo512, jnp.zeros(16, jnp.int32)])
    return base.reshape(-1), xv.reshape(-1), yv.reshape(-1), lo512_pad


def _make_remap():
    mesh = plsc.VectorSubcoreMesh(core_axis_name="c", subcore_axis_name="s")

    @functools.partial(
        pl.kernel,
        out_type=jax.ShapeDtypeStruct((_C * _N * _N,), jnp.float32),
        mesh=mesh,
        compiler_params=pltpu.CompilerParams(needs_layout_passes=False),
        scratch_types=[
            pltpu.VMEM((_PX,), jnp.int32),
            pltpu.VMEM((_PX,), jnp.float32),
            pltpu.VMEM((_PX,), jnp.float32),
            pltpu.VMEM((_NBANDS + 16,), jnp.int32),
            pltpu.VMEM((2 * _WPX,), jnp.float32),
            pltpu.VMEM((2 * _PX,), jnp.float32),
            pltpu.SemaphoreType.DMA,
            pltpu.SemaphoreType.DMA,
            pltpu.SemaphoreType.DMA,
            pltpu.SemaphoreType.DMA,
        ],
    )
    def _remap(img_hbm, base_hbm, xv_hbm, yv_hbm, lo_hbm, out_hbm,
               base_v, xv_v, yv_v, lo_v, win_v, out_v,
               sem_in0, sem_in1, sem_out0, sem_out1):
        cid = lax.axis_index("c")
        sid = lax.axis_index("s")
        wid = sid * _NC + cid
        mband = pl.multiple_of(wid * _PX, _PX)

        pltpu.sync_copy(base_hbm.at[pl.ds(mband, _PX)], base_v)
        pltpu.sync_copy(xv_hbm.at[pl.ds(mband, _PX)], xv_v)
        pltpu.sync_copy(yv_hbm.at[pl.ds(mband, _PX)], yv_v)
        pltpu.sync_copy(lo_hbm, lo_v)
        lo512 = pl.multiple_of(lo_v[pl.ds(wid, 16)][0], _N)

        sems_in = (sem_in0, sem_in1)
        sems_out = (sem_out0, sem_out1)

        def in_copy(c, p):
            return pltpu.make_async_copy(
                img_hbm.at[pl.ds(pl.multiple_of(c * (_N * _N) + lo512, _N),
                                 _WPX)],
                win_v.at[pl.ds(p * _WPX, _WPX)],
                sems_in[p])

        def out_copy(c, p):
            return pltpu.make_async_copy(
                out_v.at[pl.ds(p * _PX, _PX)],
                out_hbm.at[pl.ds(pl.multiple_of(c * (_N * _N) + mband, _PX),
                                 _PX)],
                sems_out[p])

        in_copy(0, 0).start()

        def compute(p):
            woff = p * _WPX

            @plsc.parallel_loop(0, _PX, step=16, unroll=8)
            def px(i):
                sl = pl.ds(i, 16)
                b = base_v[sl] + woff
                xv = xv_v[sl]
                yv = yv_v[sl]
                g00 = plsc.load_gather(win_v, [b])
                g01 = plsc.load_gather(win_v, [b + 1])
                g10 = plsc.load_gather(win_v, [b + _N])
                g11 = plsc.load_gather(win_v, [b + (_N + 1)])
                top = g00 + xv * (g01 - g00)
                bot = g10 + xv * (g11 - g10)
                out_v[pl.ds(p * _PX + i, 16)] = top + yv * (bot - top)

        def chan_pair(c2, _):
            for p in (0, 1):
                c = c2 * 2 + p
                in_copy(c, p).wait()

                @pl.when(c < _C - 1)
                def _start_next():
                    in_copy(c + 1, 1 - p).start()

                @pl.when(c >= 2)
                def _free_out():
                    out_copy(c - 2, p).wait()

                compute(p)
                out_copy(c, p).start()
            return 0

        lax.fori_loop(0, _C // 2, chan_pair, 0)
        out_copy(_C - 2, 0).wait()
        out_copy(_C - 1, 1).wait()

    return _remap


_REMAP_CACHE = []


def kernel(img):
    base, xv, yv, lo512 = _consts()
    if not _REMAP_CACHE:
        _REMAP_CACHE.append(_make_remap())
    img1 = img.reshape(-1)
    out = _REMAP_CACHE[0](img1, base, xv, yv, lo512)
    return out.reshape(img.shape)


# speedup vs baseline: 187.4513x; 1.0426x over previous
import functools
import math

import jax
import jax.numpy as jnp
from jax import lax
from jax.experimental import pallas as pl
from jax.experimental.pallas import tpu as pltpu
from jax.experimental.pallas import tpu_sc as plsc

_N = 512
_C = 96
_NBANDS = 32
_BAND = _N // _NBANDS
_PX = _BAND * _N
_W = 44
_WPX = _W * _N
_NC = 2
_NS = 16

_CUTMIN, _CUTMAX, _ALPHA = 2, 32, 1.0


def _build_field():
    n = _N
    beta_sample = 0.5
    cut = int(beta_sample * (_CUTMAX + 1 - _CUTMIN) + _CUTMIN)
    c_ = cut + 1e-06
    lg = math.log(c_)
    t1 = 1.0 / (math.pi * n ** 2 * lg)
    t2 = 4.0 / (math.pi ** 3 * c_ ** 2 * lg)
    t2 = max(t1, _ALPHA * t2)
    t = beta_sample * (t2 - t1) + t1

    def field(m, key):
        x = jnp.linspace(0.0, 1.0, n, dtype=jnp.float32)
        k = jnp.arange(1, m + 1, dtype=jnp.float32)
        i, j = jnp.meshgrid(k, k, indexing='ij')
        r = jnp.sqrt(i ** 2 + j ** 2)
        e = (r < m + 0.5).astype(jnp.float32) / r
        s = jnp.sin(jnp.pi * x[:, None] * k[None, :])
        c = jax.random.normal(key, (m, m), dtype=jnp.float32) * e
        return jnp.einsum('ij,xi,yj->yx', c, s, s)

    ku, kv = jax.random.split(jax.random.key(1))
    dx = (t ** 0.5) * field(cut, ku) * n
    dy = (t ** 0.5) * field(cut, kv) * n
    y, x = jnp.meshgrid(jnp.arange(n, dtype=jnp.float32),
                        jnp.arange(n, dtype=jnp.float32), indexing='ij')
    xn = jnp.clip(x - dx, 0.0, n - 1)
    yn = jnp.clip(y - dy, 0.0, n - 1)
    ybf = jnp.clip(jnp.floor(yn), 0, n - 2)
    xbf = jnp.clip(jnp.floor(xn), 0, n - 2)
    yv = yn - ybf
    xv = xn - xbf
    gbase = ybf.astype(jnp.int32) * n + xbf.astype(jnp.int32)
    return gbase, xv, yv


def _consts():
    gbase, xv, yv = _build_field()
    ybrow = gbase // _N
    ybmin = jnp.min(ybrow.reshape(_NBANDS, _PX), axis=1)
    lo = jnp.clip(ybmin - 2, 0, _N - _W).astype(jnp.int32)
    lo512 = lo * _N
    base = gbase.reshape(_NBANDS, _PX) - lo512[:, None]
    xq = jnp.round(xv * 65535.0).astype(jnp.uint32)
    yq = jnp.round(yv * 65535.0).astype(jnp.uint32)
    uv = jax.lax.bitcast_convert_type((yq << 16) | xq, jnp.int32)
    lo512_pad = jnp.concatenate([lo512, jnp.zeros(16, jnp.int32)])
    return base.reshape(-1), uv.reshape(-1), lo512_pad


def _make_remap():
    mesh = plsc.VectorSubcoreMesh(core_axis_name="c", subcore_axis_name="s")

    @functools.partial(
        pl.kernel,
        out_type=jax.ShapeDtypeStruct((_C * _N * _N,), jnp.float32),
        mesh=mesh,
        compiler_params=pltpu.CompilerParams(needs_layout_passes=False),
        scratch_types=[
            pltpu.VMEM((_PX,), jnp.int32),
            pltpu.VMEM((_PX,), jnp.int32),
            pltpu.VMEM((_NBANDS + 16,), jnp.int32),
            pltpu.VMEM((2 * _WPX,), jnp.float32),
            pltpu.VMEM((2 * _PX,), jnp.float32),
            pltpu.SemaphoreType.DMA,
            pltpu.SemaphoreType.DMA,
            pltpu.SemaphoreType.DMA,
            pltpu.SemaphoreType.DMA,
        ],
    )
    def _remap(img_hbm, base_hbm, uv_hbm, lo_hbm, out_hbm,
               base_v, uv_v, lo_v, win_v, out_v,
               sem_in0, sem_in1, sem_out0, sem_out1):
        cid = lax.axis_index("c")
        sid = lax.axis_index("s")
        wid = sid * _NC + cid
        mband = pl.multiple_of(wid * _PX, _PX)

        pltpu.sync_copy(base_hbm.at[pl.ds(mband, _PX)], base_v)
        pltpu.sync_copy(uv_hbm.at[pl.ds(mband, _PX)], uv_v)
        pltpu.sync_copy(lo_hbm, lo_v)
        lo512 = pl.multiple_of(lo_v[pl.ds(wid, 16)][0], _N)

        sems_in = (sem_in0, sem_in1)
        sems_out = (sem_out0, sem_out1)

        def in_copy(c, p):
            return pltpu.make_async_copy(
                img_hbm.at[pl.ds(pl.multiple_of(c * (_N * _N) + lo512, _N),
                                 _WPX)],
                win_v.at[pl.ds(p * _WPX, _WPX)],
                sems_in[p])

        def out_copy(c, p):
            return pltpu.make_async_copy(
                out_v.at[pl.ds(p * _PX, _PX)],
                out_hbm.at[pl.ds(pl.multiple_of(c * (_N * _N) + mband, _PX),
                                 _PX)],
                sems_out[p])

        in_copy(0, 0).start()

        def compute(p):
            woff = p * _WPX

            @plsc.parallel_loop(0, _PX, step=16, unroll=8)
            def px(i):
                sl = pl.ds(i, 16)
                b = base_v[sl] + woff
                uvw = uv_v[sl]
                xv = (uvw & 0xFFFF).astype(jnp.float32) * (1.0 / 65535.0)
                yv = (lax.shift_right_logical(uvw, 16)
                      .astype(jnp.float32) * (1.0 / 65535.0))
                g00 = plsc.load_gather(win_v, [b])
                g01 = plsc.load_gather(win_v, [b + 1])
                g10 = plsc.load_gather(win_v, [b + _N])
                g11 = plsc.load_gather(win_v, [b + (_N + 1)])
                top = g00 + xv * (g01 - g00)
                bot = g10 + xv * (g11 - g10)
                out_v[pl.ds(p * _PX + i, 16)] = top + yv * (bot - top)

        def chan_pair(c2, _):
            for p in (0, 1):
                c = c2 * 2 + p
                in_copy(c, p).wait()

                @pl.when(c < _C - 1)
                def _start_next():
                    in_copy(c + 1, 1 - p).start()

                @pl.when(c >= 2)
                def _free_out():
                    out_copy(c - 2, p).wait()

                compute(p)
                out_copy(c, p).start()
            return 0

        lax.fori_loop(0, _C // 2, chan_pair, 0)
        out_copy(_C - 2, 0).wait()
        out_copy(_C - 1, 1).wait()

    return _remap


_REMAP_CACHE = []


def kernel(img):
    base, uv, lo512 = _consts()
    if not _REMAP_CACHE:
        _REMAP_CACHE.append(_make_remap())
    img1 = img.reshape(-1)
    out = _REMAP_CACHE[0](img1, base, uv, lo512)
    return out.reshape(img.shape)
